# baseline (speedup 1.0000x reference)
import functools

import jax
import jax.numpy as jnp
from jax import lax
from jax.experimental import pallas as pl
from jax.experimental.pallas import tpu as pltpu
from jax.experimental.pallas import tpu_sc as plsc

N = 10000
D = 128
E = 320000
NLAYERS = 3

NC = 2
NS = 16
DH = D // NC

RPT = 626
N_PAD = NS * RPT
FL = N // NS

BLK = 40
EPT = E // NS
NBLK = EPT // BLK

_mesh = plsc.VectorSubcoreMesh(core_axis_name="c", subcore_axis_name="s")

_CHUNKS = []
_off = 0
while _off < RPT:
  _sz = min(BLK, RPT - _off)
  _CHUNKS.append((_off, _sz))
  _off += _sz


def _zero_rows(buf, nrows):
  zero = jnp.zeros((16,), jnp.float32)

  def zrow(i, carry):
    for j in range(DH // 16):
      buf[i, pl.ds(j * 16, 16)] = zero
    return carry

  lax.fori_loop(0, nrows, zrow, 0)


def _relu_rows(buf, nrows):
  zero = jnp.zeros((16,), jnp.float32)

  def rrow(i, carry):
    for j in range(DH // 16):
      buf[i, pl.ds(j * 16, 16)] = jnp.maximum(buf[i, pl.ds(j * 16, 16)], zero)
    return carry

  lax.fori_loop(0, nrows, rrow, 0)


@functools.partial(
    pl.kernel,
    out_type=jax.ShapeDtypeStruct((N, D), jnp.float32),
    mesh=_mesh,
    scratch_types=[
        pltpu.VMEM((NBLK, BLK), jnp.int32),
        pltpu.VMEM((NBLK, BLK), jnp.int32),
        pltpu.VMEM((2, BLK, DH), jnp.float32),
        pltpu.VMEM_SHARED((N_PAD, DH), jnp.float32),
        pltpu.VMEM_SHARED((N_PAD, DH), jnp.float32),
        pltpu.SemaphoreType.DMA,
        pltpu.SemaphoreType.DMA,
        pltpu.SemaphoreType.DMA,
        pltpu.SemaphoreType.DMA,
    ],
    compiler_params=pltpu.CompilerParams(use_tc_tiling_on_sc=False),
)
def _gcn_sc(f_hbm, src_hbm, dst_hbm, out_hbm, sidx, didx, gbuf, T0, T1,
            g0, g1, s0, s1):
  c = lax.axis_index("c")
  s = lax.axis_index("s")
  rbase = s * RPT

  pltpu.sync_copy(src_hbm.at[s], sidx)
  pltpu.sync_copy(dst_hbm.at[s], didx)
  pltpu.sync_copy(f_hbm.at[pl.ds(s * FL, FL), pl.ds(c * DH, DH)],
                  T0.at[pl.ds(s * FL, FL)])
  _zero_rows(gbuf.at[0], BLK)
  for off, sz in _CHUNKS:
    pltpu.sync_copy(gbuf.at[0, pl.ds(0, sz)], T1.at[pl.ds(rbase + off, sz)])
  plsc.subcore_barrier()

  for layer in range(NLAYERS):
    Ts = T0 if layer % 2 == 0 else T1
    Td = T1 if layer % 2 == 0 else T0

    pltpu.async_copy(Ts.at[sidx.at[0]], gbuf.at[0], g0)
    pltpu.async_copy(Ts.at[sidx.at[1]], gbuf.at[1], g1)

    def pair(i, carry):
      j = 2 * i

      pltpu.make_async_copy(Ts.at[sidx.at[j]], gbuf.at[0], g0).wait()
      pltpu.async_copy(gbuf.at[0], Td.at[didx.at[j]], s0, add=True)

      pltpu.make_async_copy(Ts.at[sidx.at[j + 1]], gbuf.at[1], g1).wait()
      pltpu.async_copy(gbuf.at[1], Td.at[didx.at[j + 1]], s1, add=True)

      @pl.when(j + 2 < NBLK)
      def _():
        pltpu.make_async_copy(gbuf.at[0], Td.at[didx.at[j]], s0).wait()
        pltpu.async_copy(Ts.at[sidx.at[j + 2]], gbuf.at[0], g0)

      @pl.when(j + 3 < NBLK)
      def _():
        pltpu.make_async_copy(gbuf.at[1], Td.at[didx.at[j + 1]], s1).wait()
        pltpu.async_copy(Ts.at[sidx.at[j + 3]], gbuf.at[1], g1)

      return carry

    lax.fori_loop(0, NBLK // 2, pair, 0)
    pltpu.make_async_copy(gbuf.at[0], Td.at[didx.at[0]], s0).wait()
    pltpu.make_async_copy(gbuf.at[1], Td.at[didx.at[1]], s1).wait()
    plsc.subcore_barrier()

    if layer < NLAYERS - 1:
      _zero_rows(gbuf.at[1], BLK)
      for off, sz in _CHUNKS:
        pltpu.sync_copy(Td.at[pl.ds(rbase + off, sz)], gbuf.at[0, pl.ds(0, sz)])
        _relu_rows(gbuf.at[0], sz)
        pltpu.sync_copy(gbuf.at[0, pl.ds(0, sz)], Td.at[pl.ds(rbase + off, sz)])
        pltpu.sync_copy(gbuf.at[1, pl.ds(0, sz)], Ts.at[pl.ds(rbase + off, sz)])
      plsc.subcore_barrier()
    else:
      pltpu.sync_copy(Td.at[pl.ds(s * FL, FL)],
                      out_hbm.at[pl.ds(s * FL, FL), pl.ds(c * DH, DH)])


def kernel(features, edge_index):
  src = edge_index[0].astype(jnp.int32).reshape(NS, NBLK, BLK)
  dst = edge_index[1].astype(jnp.int32).reshape(NS, NBLK, BLK)
  return _gcn_sc(features, src, dst)


# speedup vs baseline: 1.0647x; 1.0647x over previous
import functools

import jax
import jax.numpy as jnp
from jax import lax
from jax.experimental import pallas as pl
from jax.experimental.pallas import tpu as pltpu
from jax.experimental.pallas import tpu_sc as plsc

N = 10000
D = 128
E = 320000
NLAYERS = 3

NC = 2
NS = 16
DH = D // NC

RPT = 626
N_PAD = NS * RPT
FL = N // NS

BLK = 64
EPT = E // NS
NBLK = 314
E_PAD = NS * NBLK * BLK

SRC_PAD_ROW = N + 2
DST_PAD_ROW = N + 1

_mesh = plsc.VectorSubcoreMesh(core_axis_name="c", subcore_axis_name="s")

_CHUNKS = []
_off = 0
while _off < RPT:
  _sz = min(BLK, RPT - _off)
  _CHUNKS.append((_off, _sz))
  _off += _sz


def _zero_rows(buf, nrows):
  zero = jnp.zeros((16,), jnp.float32)

  def zrow(i, carry):
    for j in range(DH // 16):
      buf[i, pl.ds(j * 16, 16)] = zero
    return carry

  lax.fori_loop(0, nrows, zrow, 0)


def _relu_rows(buf, nrows):
  zero = jnp.zeros((16,), jnp.float32)

  def rrow(i, carry):
    for j in range(DH // 16):
      buf[i, pl.ds(j * 16, 16)] = jnp.maximum(buf[i, pl.ds(j * 16, 16)], zero)
    return carry

  lax.fori_loop(0, nrows, rrow, 0)


@functools.partial(
    pl.kernel,
    out_type=jax.ShapeDtypeStruct((N, D), jnp.float32),
    mesh=_mesh,
    scratch_types=[
        pltpu.VMEM((NBLK, BLK), jnp.int32),
        pltpu.VMEM((NBLK, BLK), jnp.int32),
        pltpu.VMEM((2, BLK, DH), jnp.float32),
        pltpu.VMEM_SHARED((N_PAD, DH), jnp.float32),
        pltpu.VMEM_SHARED((N_PAD, DH), jnp.float32),
        pltpu.SemaphoreType.DMA,
        pltpu.SemaphoreType.DMA,
        pltpu.SemaphoreType.DMA,
        pltpu.SemaphoreType.DMA,
    ],
    compiler_params=pltpu.CompilerParams(use_tc_tiling_on_sc=False),
)
def _gcn_sc(f_hbm, src_hbm, dst_hbm, out_hbm, sidx, didx, gbuf, T0, T1,
            g0, g1, s0, s1):
  c = lax.axis_index("c")
  s = lax.axis_index("s")
  rbase = s * RPT

  pltpu.sync_copy(src_hbm.at[s], sidx)
  pltpu.sync_copy(dst_hbm.at[s], didx)
  pltpu.sync_copy(f_hbm.at[pl.ds(s * FL, FL), pl.ds(c * DH, DH)],
                  T0.at[pl.ds(s * FL, FL)])
  _zero_rows(gbuf.at[0], BLK)
  for off, sz in _CHUNKS:
    pltpu.sync_copy(gbuf.at[0, pl.ds(0, sz)], T1.at[pl.ds(rbase + off, sz)])

  @pl.when(s == 0)
  def _():
    pltpu.sync_copy(gbuf.at[0, pl.ds(0, N_PAD - N)], T0.at[pl.ds(N, N_PAD - N)])

  plsc.subcore_barrier()

  for layer in range(NLAYERS):
    Ts = T0 if layer % 2 == 0 else T1
    Td = T1 if layer % 2 == 0 else T0

    pltpu.async_copy(Ts.at[sidx.at[0]], gbuf.at[0], g0)
    pltpu.async_copy(Ts.at[sidx.at[1]], gbuf.at[1], g1)

    def pair(i, carry):
      j = 2 * i

      pltpu.make_async_copy(Ts.at[sidx.at[j]], gbuf.at[0], g0).wait()
      pltpu.async_copy(gbuf.at[0], Td.at[didx.at[j]], s0, add=True)

      pltpu.make_async_copy(Ts.at[sidx.at[j + 1]], gbuf.at[1], g1).wait()
      pltpu.async_copy(gbuf.at[1], Td.at[didx.at[j + 1]], s1, add=True)

      @pl.when(j + 2 < NBLK)
      def _():
        pltpu.make_async_copy(gbuf.at[0], Td.at[didx.at[j]], s0).wait()
        pltpu.async_copy(Ts.at[sidx.at[j + 2]], gbuf.at[0], g0)

      @pl.when(j + 3 < NBLK)
      def _():
        pltpu.make_async_copy(gbuf.at[1], Td.at[didx.at[j + 1]], s1).wait()
        pltpu.async_copy(Ts.at[sidx.at[j + 3]], gbuf.at[1], g1)

      return carry

    lax.fori_loop(0, NBLK // 2, pair, 0)
    pltpu.make_async_copy(gbuf.at[0], Td.at[didx.at[0]], s0).wait()
    pltpu.make_async_copy(gbuf.at[1], Td.at[didx.at[1]], s1).wait()
    plsc.subcore_barrier()

    if layer < NLAYERS - 1:
      _zero_rows(gbuf.at[1], BLK)
      for off, sz in _CHUNKS:
        pltpu.sync_copy(Td.at[pl.ds(rbase + off, sz)], gbuf.at[0, pl.ds(0, sz)])
        _relu_rows(gbuf.at[0], sz)
        pltpu.sync_copy(gbuf.at[0, pl.ds(0, sz)], Td.at[pl.ds(rbase + off, sz)])
        pltpu.sync_copy(gbuf.at[1, pl.ds(0, sz)], Ts.at[pl.ds(rbase + off, sz)])
      plsc.subcore_barrier()
    else:
      pltpu.sync_copy(Td.at[pl.ds(s * FL, FL)],
                      out_hbm.at[pl.ds(s * FL, FL), pl.ds(c * DH, DH)])


def kernel(features, edge_index):
  src = edge_index[0].astype(jnp.int32)
  dst = edge_index[1].astype(jnp.int32)
  pad = E_PAD - E
  src = jnp.concatenate([src, jnp.full((pad,), SRC_PAD_ROW, jnp.int32)])
  dst = jnp.concatenate([dst, jnp.full((pad,), DST_PAD_ROW, jnp.int32)])
  return _gcn_sc(features, src.reshape(NS, NBLK, BLK),
                 dst.reshape(NS, NBLK, BLK))


# speedup vs baseline: 1.0860x; 1.0200x over previous
import functools

import jax
import jax.numpy as jnp
from jax import lax
from jax.experimental import pallas as pl
from jax.experimental.pallas import tpu as pltpu
from jax.experimental.pallas import tpu_sc as plsc

N = 10000
D = 128
E = 320000
NLAYERS = 3

NC = 2
NS = 16
DH = D // NC

RPT = 632
N_PAD = NS * RPT
FL = N // NS

BLK = 64
EPT = -(-E // NS)
NBLK = -(-EPT // BLK)
E_PAD = NS * NBLK * BLK

SRC_PAD_ROW = N + 1
DST_PAD_ROW = N

_mesh = plsc.VectorSubcoreMesh(core_axis_name="c", subcore_axis_name="s")

_CHUNKS = []
_off = 0
while _off < RPT:
  _sz = min(BLK, RPT - _off)
  _CHUNKS.append((_off, _sz))
  _off += _sz


def _zero_rows(buf, nrows):
  zero = jnp.zeros((16,), jnp.float32)

  def zrow(i, carry):
    for j in range(DH // 16):
      buf[i, pl.ds(j * 16, 16)] = zero
    return carry

  lax.fori_loop(0, nrows, zrow, 0)


def _relu_rows(buf, nrows):
  zero = jnp.zeros((16,), jnp.float32)

  def rrow(i, carry):
    for j in range(DH // 16):
      buf[i, pl.ds(j * 16, 16)] = jnp.maximum(buf[i, pl.ds(j * 16, 16)], zero)
    return carry

  lax.fori_loop(0, nrows, rrow, 0)


@functools.partial(
    pl.kernel,
    out_type=jax.ShapeDtypeStruct((N, D), jnp.float32),
    mesh=_mesh,
    scratch_types=[
        pltpu.VMEM((NBLK, BLK), jnp.int32),
        pltpu.VMEM((NBLK, BLK), jnp.int32),
        pltpu.VMEM((2, BLK, DH), jnp.float32),
        pltpu.VMEM_SHARED((N_PAD, DH), jnp.float32),
        pltpu.VMEM_SHARED((N_PAD, DH), jnp.float32),
        pltpu.SemaphoreType.DMA,
        pltpu.SemaphoreType.DMA,
    ],
    compiler_params=pltpu.CompilerParams(use_tc_tiling_on_sc=False),
)
def _gcn_sc(f_hbm, src_hbm, dst_hbm, out_hbm, sidx, didx, gbuf, A, B,
            gsem, ssem):
  c = lax.axis_index("c")
  s = lax.axis_index("s")
  rbase = s * RPT

  pltpu.sync_copy(src_hbm.at[s], sidx)
  pltpu.sync_copy(dst_hbm.at[s], didx)
  pltpu.sync_copy(f_hbm.at[pl.ds(s * FL, FL), pl.ds(c * DH, DH)],
                  A.at[pl.ds(s * FL, FL)])
  _zero_rows(gbuf.at[0], BLK)
  for off, sz in _CHUNKS:
    pltpu.sync_copy(gbuf.at[0, pl.ds(0, sz)], B.at[pl.ds(rbase + off, sz)])

  @pl.when(s == 0)
  def _():
    pltpu.sync_copy(gbuf.at[0, pl.ds(0, BLK)], A.at[pl.ds(N, BLK)])
    pltpu.sync_copy(gbuf.at[0, pl.ds(0, N_PAD - N - BLK)],
                    A.at[pl.ds(N + BLK, N_PAD - N - BLK)])

  plsc.subcore_barrier()

  for layer in range(NLAYERS):
    pltpu.async_copy(A.at[sidx.at[0]], gbuf.at[0], gsem)

    def step(j, carry):
      b = lax.rem(j, 2)
      nb = lax.rem(j + 1, 2)

      @pl.when(j >= 1)
      def _():
        pltpu.make_async_copy(gbuf.at[nb], B.at[didx.at[j - 1]], ssem).wait()

      @pl.when(j + 1 < NBLK)
      def _():
        pltpu.async_copy(A.at[sidx.at[j + 1]], gbuf.at[nb], gsem)

      pltpu.make_async_copy(A.at[sidx.at[j]], gbuf.at[b], gsem).wait()
      pltpu.async_copy(gbuf.at[b], B.at[didx.at[j]], ssem, add=True)
      return carry

    lax.fori_loop(0, NBLK, step, 0)
    lastb = (NBLK - 1) % 2
    pltpu.make_async_copy(
        gbuf.at[lastb], B.at[didx.at[NBLK - 1]], ssem).wait()
    plsc.subcore_barrier()

    if layer < NLAYERS - 1:
      for off, sz in _CHUNKS:
        pltpu.sync_copy(B.at[pl.ds(rbase + off, sz)], gbuf.at[0, pl.ds(0, sz)])
        _relu_rows(gbuf.at[0], sz)
        pltpu.sync_copy(gbuf.at[0, pl.ds(0, sz)], A.at[pl.ds(rbase + off, sz)])
        _zero_rows(gbuf.at[0], sz)
        pltpu.sync_copy(gbuf.at[0, pl.ds(0, sz)], B.at[pl.ds(rbase + off, sz)])
      plsc.subcore_barrier()
    else:
      pltpu.sync_copy(B.at[pl.ds(s * FL, FL)],
                      out_hbm.at[pl.ds(s * FL, FL), pl.ds(c * DH, DH)])


def kernel(features, edge_index):
  src = edge_index[0].astype(jnp.int32)
  dst = edge_index[1].astype(jnp.int32)
  pad = E_PAD - E
  src = jnp.concatenate([src, jnp.full((pad,), SRC_PAD_ROW, jnp.int32)])
  dst = jnp.concatenate([dst, jnp.full((pad,), DST_PAD_ROW, jnp.int32)])
  return _gcn_sc(features, src.reshape(NS, NBLK, BLK),
                 dst.reshape(NS, NBLK, BLK))


# speedup vs baseline: 1.1283x; 1.0389x over previous
import functools

import jax
import jax.numpy as jnp
from jax import lax
from jax.experimental import pallas as pl
from jax.experimental.pallas import tpu as pltpu
from jax.experimental.pallas import tpu_sc as plsc

N = 10000
D = 128
E = 320000
NLAYERS = 3

NC = 2
NS = 16
DH = D // NC

RPT = 632
N_PAD = NS * RPT
FL = N // NS

BLK = 64
EPT = -(-E // NS)
NBLK = -(-EPT // BLK)
E_PAD = NS * NBLK * BLK

SRC_PAD_ROW = N + 1
DST_PAD_ROW = N

_mesh = plsc.VectorSubcoreMesh(core_axis_name="c", subcore_axis_name="s")

_CHUNKS = []
_off = 0
while _off < RPT:
  _sz = min(BLK, RPT - _off)
  _CHUNKS.append((_off, _sz))
  _off += _sz

HS = BLK // 2
_CHUNKS32 = []
_off = 0
while _off < RPT:
  _sz = min(HS, RPT - _off)
  _CHUNKS32.append((_off, _sz))
  _off += _sz


def _zero_rows(buf, nrows):
  zero = jnp.zeros((16,), jnp.float32)

  def zrow(i, carry):
    for j in range(DH // 16):
      buf[i, pl.ds(j * 16, 16)] = zero
    return carry

  lax.fori_loop(0, nrows, zrow, 0)


def _relu_rows(buf, base, nrows):
  zero = jnp.zeros((16,), jnp.float32)

  def rrow(i, carry):
    for j in range(DH // 16):
      buf[base + i, pl.ds(j * 16, 16)] = jnp.maximum(
          buf[base + i, pl.ds(j * 16, 16)], zero)
    return carry

  lax.fori_loop(0, nrows, rrow, 0)


@functools.partial(
    pl.kernel,
    out_type=jax.ShapeDtypeStruct((N, D), jnp.float32),
    mesh=_mesh,
    scratch_types=[
        pltpu.VMEM((NBLK, BLK), jnp.int32),
        pltpu.VMEM((NBLK, BLK), jnp.int32),
        pltpu.VMEM((2, BLK, DH), jnp.float32),
        pltpu.VMEM_SHARED((N_PAD, DH), jnp.float32),
        pltpu.VMEM_SHARED((N_PAD, DH), jnp.float32),
        pltpu.SemaphoreType.DMA,
        pltpu.SemaphoreType.DMA,
        pltpu.SemaphoreType.DMA,
        pltpu.SemaphoreType.DMA,
        pltpu.SemaphoreType.DMA,
    ],
    compiler_params=pltpu.CompilerParams(use_tc_tiling_on_sc=False),
)
def _gcn_sc(f_hbm, src_hbm, dst_hbm, out_hbm, sidx, didx, gbuf, A, B,
            gsem, ssem, osem, zsem, fsem):
  c = lax.axis_index("c")
  s = lax.axis_index("s")
  rbase = s * RPT

  cp_si = pltpu.async_copy(src_hbm.at[s], sidx, gsem)
  cp_di = pltpu.async_copy(dst_hbm.at[s], didx, ssem)
  cp_f = pltpu.async_copy(f_hbm.at[pl.ds(s * FL, FL), pl.ds(c * DH, DH)],
                          A.at[pl.ds(s * FL, FL)], fsem)
  _zero_rows(gbuf.at[0], BLK)
  for off, sz in _CHUNKS:
    pltpu.async_copy(gbuf.at[0, pl.ds(0, sz)], B.at[pl.ds(rbase + off, sz)],
                     zsem)

  @pl.when(s == 0)
  def _():
    pltpu.sync_copy(gbuf.at[0, pl.ds(0, BLK)], A.at[pl.ds(N, BLK)])
    pltpu.sync_copy(gbuf.at[0, pl.ds(0, N_PAD - N - BLK)],
                    A.at[pl.ds(N + BLK, N_PAD - N - BLK)])

  cp_si.wait()
  cp_di.wait()
  cp_f.wait()
  for off, sz in _CHUNKS:
    pltpu.make_async_copy(gbuf.at[0, pl.ds(0, sz)],
                          B.at[pl.ds(rbase + off, sz)], zsem).wait()
  plsc.subcore_barrier()

  for layer in range(NLAYERS):
    pltpu.async_copy(A.at[sidx.at[0]], gbuf.at[0], gsem)

    def step(j, carry):
      b = lax.rem(j, 2)
      nb = lax.rem(j + 1, 2)

      @pl.when(j >= 1)
      def _():
        pltpu.make_async_copy(gbuf.at[nb], B.at[didx.at[j - 1]], ssem).wait()

      @pl.when(j + 1 < NBLK)
      def _():
        pltpu.async_copy(A.at[sidx.at[j + 1]], gbuf.at[nb], gsem)

      pltpu.make_async_copy(A.at[sidx.at[j]], gbuf.at[b], gsem).wait()
      pltpu.async_copy(gbuf.at[b], B.at[didx.at[j]], ssem, add=True)
      return carry

    lax.fori_loop(0, NBLK, step, 0)
    lastb = (NBLK - 1) % 2
    pltpu.make_async_copy(
        gbuf.at[lastb], B.at[didx.at[NBLK - 1]], ssem).wait()
    plsc.subcore_barrier()

    if layer < NLAYERS - 1:
      _zero_rows(gbuf.at[1], HS)
      K = len(_CHUNKS32)

      def b_strip(k):
        off, sz = _CHUNKS32[k]
        return B.at[pl.ds(rbase + off, sz)]

      def a_strip(k):
        off, sz = _CHUNKS32[k]
        return A.at[pl.ds(rbase + off, sz)]

      def panel(k):
        return gbuf.at[0, pl.ds((k % 2) * HS, _CHUNKS32[k][1])]

      pltpu.async_copy(b_strip(0), panel(0), gsem)
      for k in range(K):
        pltpu.make_async_copy(b_strip(k), panel(k), gsem).wait()
        pltpu.async_copy(gbuf.at[1, pl.ds(0, _CHUNKS32[k][1])], b_strip(k),
                         zsem)
        if k + 1 < K:
          if k >= 1:
            pltpu.make_async_copy(panel(k - 1), a_strip(k - 1), osem).wait()
          pltpu.async_copy(b_strip(k + 1), panel(k + 1), gsem)
        _relu_rows(gbuf.at[0], (k % 2) * HS, _CHUNKS32[k][1])
        pltpu.async_copy(panel(k), a_strip(k), osem)
      for k in range(max(0, K - 2), K):
        pltpu.make_async_copy(panel(k), a_strip(k), osem).wait()
      for k in range(K):
        pltpu.make_async_copy(gbuf.at[1, pl.ds(0, _CHUNKS32[k][1])],
                              b_strip(k), zsem).wait()
      plsc.subcore_barrier()
    else:
      pltpu.sync_copy(B.at[pl.ds(s * FL, FL)],
                      out_hbm.at[pl.ds(s * FL, FL), pl.ds(c * DH, DH)])


def kernel(features, edge_index):
  src = edge_index[0].astype(jnp.int32)
  dst = edge_index[1].astype(jnp.int32)
  pad = E_PAD - E
  src = jnp.concatenate([src, jnp.full((pad,), SRC_PAD_ROW, jnp.int32)])
  dst = jnp.concatenate([dst, jnp.full((pad,), DST_PAD_ROW, jnp.int32)])
  return _gcn_sc(features, src.reshape(NS, NBLK, BLK),
                 dst.reshape(NS, NBLK, BLK))
